# Initial kernel scaffold; baseline (speedup 1.0000x reference)
import jax
import jax.numpy as jnp
from jax.experimental import pallas as pl


def kernel(x, W):
    raise NotImplementedError("write your pallas kernel here")


# speedup vs baseline: 2.3009x; 2.3009x over previous
import functools

import jax
import jax.numpy as jnp
from jax import lax
from jax.experimental import pallas as pl
from jax.experimental.pallas import tpu as pltpu
from jax.experimental.pallas import tpu_sc as plsc

BLANK_TOKEN_ID = 0
NC = 2
NS = 16
L = 16

BATCH = 4
SEQ = 8192
D = 1024
TOKENS = BATCH * SEQ
NW = NC * NS
CHUNK = TOKENS // NW
CHUNKS_PER_ROW = SEQ // CHUNK
C = 32
NSTEP = CHUNK // C
NG = NSTEP // 2


def _body(x_hbm, w_hbm, out_hbm, xrow, idx, rows0, rows1, gsem0, gsem1):
    c = lax.axis_index("c")
    s = lax.axis_index("s")
    wid = c * NS + s
    b = wid // CHUNKS_PER_ROW
    k = wid % CHUNKS_PER_ROW

    pltpu.sync_copy(x_hbm.at[pl.ds(b * SEQ, SEQ)], xrow)

    def count_body(i, carry):
        v = xrow[pl.ds(i * L, L)]
        isb = jnp.where(v == BLANK_TOKEN_ID, jnp.int32(1), jnp.int32(0))
        return carry + jnp.sum(isb)

    carry0 = lax.fori_loop(0, k * (CHUNK // L), count_body, jnp.int32(0))

    base0 = k * CHUNK

    def pos_body(i, carry):
        v = xrow[pl.ds(base0 + i * L, L)]
        isb = jnp.where(v == BLANK_TOKEN_ID, jnp.int32(1), jnp.int32(0))
        cs = plsc.cumsum(isb)
        posv = (base0 + i * L + lax.iota(jnp.int32, L)) - (cs + carry)
        idx[pl.ds(i * L, L)] = jnp.maximum(posv, 0)
        return carry + jnp.sum(isb)

    lax.fori_loop(0, CHUNK // L, pos_body, carry0)

    tbase = wid * CHUNK

    def gather_start(step, buf, sem):
        return pltpu.async_copy(w_hbm.at[idx.at[pl.ds(step * C, C)]], buf, sem)

    def gather_wait(step, buf, sem):
        pltpu.make_async_copy(w_hbm.at[idx.at[pl.ds(step * C, C)]], buf, sem).wait()

    def scatter(step, buf):
        pltpu.sync_copy(buf, out_hbm.at[pl.ds(tbase + step * C, C)])

    gather_start(0, rows0, gsem0)

    def grp(g, _):
        a = 2 * g
        gather_start(a + 1, rows1, gsem1)
        gather_wait(a, rows0, gsem0)
        scatter(a, rows0)

        @pl.when(g < NG - 1)
        def _prefetch():
            gather_start(a + 2, rows0, gsem0)

        gather_wait(a + 1, rows1, gsem1)
        scatter(a + 1, rows1)
        return 0

    lax.fori_loop(0, NG, grp, 0)


@jax.jit
def kernel(x, W):
    x_flat = x.reshape(TOKENS).astype(jnp.int32)
    W = W.astype(jnp.float32)

    mesh = plsc.VectorSubcoreMesh(
        core_axis_name="c", subcore_axis_name="s", num_cores=NC, num_subcores=NS
    )
    run = pl.kernel(
        _body,
        out_type=jax.ShapeDtypeStruct((TOKENS, D), jnp.float32),
        mesh=mesh,
        scratch_types=[
            pltpu.VMEM((SEQ,), jnp.int32),
            pltpu.VMEM((CHUNK,), jnp.int32),
            pltpu.VMEM((C, D), jnp.float32),
            pltpu.VMEM((C, D), jnp.float32),
            pltpu.SemaphoreType.DMA,
            pltpu.SemaphoreType.DMA,
        ],
        compiler_params=pltpu.CompilerParams(needs_layout_passes=False),
    )
    out = run(x_flat, W)
    return out.reshape(BATCH, SEQ, D)
